# baseline (speedup 1.0000x reference)
import jax
import jax.numpy as jnp
from jax import lax
from jax.experimental import pallas as pl
from jax.experimental.pallas import tpu as pltpu
from jax.experimental.pallas import tpu_sc as plsc

_N = 10000
_E = 320000
_H = 128
_HH = 64
_TD = 384
_NL = 3
_NS = 16
_NPAD = 10240
_NZ = _NPAD // _NS
_EPT = _E // _NS
_EPT2 = _E // (2 * _NS)
_DW = 800
_W = 200
_WA = 104
_WB = 96
_ZCH = 160
_RB = 1000

_f32 = jnp.float32


def _pool_body(te_ref, o_ref):
    o_ref[...] = jnp.mean(te_ref[...], axis=0, keepdims=True)


def _hyper_body(p_ref, wg_ref, o_ref):
    o_ref[0] = jnp.dot(p_ref[...], wg_ref[0],
                       preferred_element_type=_f32)


def _layer0_body(nf_ref, wp_ref, bp_ref, dego_ref, w_ref, o_ref):
    x = jnp.dot(nf_ref[...], wp_ref[...], preferred_element_type=_f32)
    x = x + bp_ref[...]
    s = lax.rsqrt(jnp.maximum(dego_ref[...], 1.0))
    o_ref[...] = jnp.dot(x * s, w_ref[...], preferred_element_type=_f32)


def _layermid_body(agg_ref, dego_ref, degi_ref, w_ref, o_ref):
    x = agg_ref[0] + agg_ref[1]
    t = lax.rsqrt(jnp.maximum(degi_ref[...], 1.0))
    s = lax.rsqrt(jnp.maximum(dego_ref[...], 1.0))
    x = jnp.maximum(x * t, 0.0) * s
    o_ref[...] = jnp.dot(x, w_ref[...], preferred_element_type=_f32)


def _head_body(agg_ref, degi_ref, w1_ref, b1_ref, w2_ref, b2_ref, o_ref):
    x = agg_ref[0] + agg_ref[1]
    t = lax.rsqrt(jnp.maximum(degi_ref[...], 1.0))
    x = x * t
    h = jnp.dot(x, w1_ref[...], preferred_element_type=_f32) + b1_ref[...]
    h = jnp.maximum(h, 0.0)
    o_ref[...] = jnp.dot(h, w2_ref[...], preferred_element_type=_f32) + b2_ref[...]


def _pool_call(te):
    return pl.pallas_call(
        _pool_body,
        out_shape=jax.ShapeDtypeStruct((1, _TD), _f32),
    )(te)


def _hyper_call(pooled, Wg):
    return pl.pallas_call(
        _hyper_body,
        grid=(_NL, 8),
        in_specs=[
            pl.BlockSpec((1, _TD), lambda l, j: (0, 0)),
            pl.BlockSpec((1, _TD, 2048), lambda l, j: (l, 0, j)),
        ],
        out_specs=pl.BlockSpec((1, 1, 2048), lambda l, j: (l, 0, j)),
        out_shape=jax.ShapeDtypeStruct((_NL, 1, _H * _H), _f32),
    )(pooled, Wg)


def _layer0_call(nf, Wp, bp, dego, W0):
    return pl.pallas_call(
        _layer0_body,
        grid=(_N // _RB,),
        in_specs=[
            pl.BlockSpec((_RB, _H), lambda i: (i, 0)),
            pl.BlockSpec((_H, _H), lambda i: (0, 0)),
            pl.BlockSpec((1, _H), lambda i: (0, 0)),
            pl.BlockSpec((_RB, 1), lambda i: (i, 0)),
            pl.BlockSpec((_H, _H), lambda i: (0, 0)),
        ],
        out_specs=pl.BlockSpec((_RB, _H), lambda i: (i, 0)),
        out_shape=jax.ShapeDtypeStruct((_N, _H), _f32),
    )(nf, Wp, bp, dego, W0)


def _layermid_call(agg, dego, degi, Wl):
    return pl.pallas_call(
        _layermid_body,
        grid=(_N // _RB,),
        in_specs=[
            pl.BlockSpec((2, _RB, _H), lambda i: (0, i, 0)),
            pl.BlockSpec((_RB, 1), lambda i: (i, 0)),
            pl.BlockSpec((_RB, 1), lambda i: (i, 0)),
            pl.BlockSpec((_H, _H), lambda i: (0, 0)),
        ],
        out_specs=pl.BlockSpec((_RB, _H), lambda i: (i, 0)),
        out_shape=jax.ShapeDtypeStruct((_N, _H), _f32),
    )(agg, dego, degi, Wl)


def _head_call(agg, degi, W1, b1, W2, b2):
    return pl.pallas_call(
        _head_body,
        grid=(_N // _RB,),
        in_specs=[
            pl.BlockSpec((2, _RB, _H), lambda i: (0, i, 0)),
            pl.BlockSpec((_RB, 1), lambda i: (i, 0)),
            pl.BlockSpec((_H, _HH), lambda i: (0, 0)),
            pl.BlockSpec((1, _HH), lambda i: (0, 0)),
            pl.BlockSpec((_HH, 1), lambda i: (0, 0)),
            pl.BlockSpec((1, 1), lambda i: (0, 0)),
        ],
        out_specs=pl.BlockSpec((_RB, 1), lambda i: (i, 0)),
        out_shape=jax.ShapeDtypeStruct((_N, 1), _f32),
    )(agg, degi, W1, b1, W2, b2)


def _vector_mesh():
    return plsc.VectorSubcoreMesh(
        core_axis_name="core", subcore_axis_name="subcore")


def _deg_call(src, dst):
    @pl.kernel(
        out_type=[jax.ShapeDtypeStruct((_NPAD,), _f32),
                  jax.ShapeDtypeStruct((_NPAD,), _f32)],
        mesh=_vector_mesh(),
        scratch_types=[
            pltpu.VMEM_SHARED((_NPAD,), _f32),
            pltpu.VMEM((1, _DW), jnp.int32),
            pltpu.VMEM((1, _DW), _f32),
        ],
    )
    def deg_kernel(src_hbm, dst_hbm, out0_hbm, out1_hbm, acc_sh, idx_v, val_v):
        c = lax.axis_index("core")
        s = lax.axis_index("subcore")

        @pl.loop(0, _DW // 16)
        def _zero(i):
            val_v[0, pl.ds(i * 16, 16)] = jnp.zeros((16,), _f32)

        pltpu.sync_copy(val_v.at[0, pl.ds(0, _NZ)],
                        acc_sh.at[pl.ds(s * _NZ, _NZ)])
        plsc.subcore_barrier()

        @pl.loop(0, _DW // 16)
        def _ones(i):
            val_v[0, pl.ds(i * 16, 16)] = jnp.ones((16,), _f32)

        @pl.loop(0, _EPT // _DW)
        def _win(k):
            off = s * _EPT + k * _DW

            @pl.when(c == 0)
            def _():
                pltpu.sync_copy(src_hbm.at[pl.ds(off, _DW)], idx_v.at[0])

            @pl.when(c == 1)
            def _():
                pltpu.sync_copy(dst_hbm.at[pl.ds(off, _DW)], idx_v.at[0])

            pltpu.sync_copy(val_v.at[0], acc_sh.at[idx_v.at[0]], add=True)

        plsc.subcore_barrier()

        @pl.when(c == 0)
        def _():
            pltpu.sync_copy(acc_sh.at[pl.ds(s * _NZ, _NZ)],
                            out0_hbm.at[pl.ds(s * _NZ, _NZ)])

        @pl.when(c == 1)
        def _():
            pltpu.sync_copy(acc_sh.at[pl.ds(s * _NZ, _NZ)],
                            out1_hbm.at[pl.ds(s * _NZ, _NZ)])

    return deg_kernel(src, dst)


def _agg_call(y, src, dst):
    @pl.kernel(
        out_type=jax.ShapeDtypeStruct((2, _NPAD, _H), _f32),
        mesh=_vector_mesh(),
        scratch_types=[
            pltpu.VMEM_SHARED((_NPAD, _H), _f32),
            pltpu.VMEM((1, _WA), jnp.int32),
            pltpu.VMEM((1, _WB), jnp.int32),
            pltpu.VMEM((1, _WA), jnp.int32),
            pltpu.VMEM((1, _WB), jnp.int32),
            pltpu.VMEM((_W, _H), _f32),
            pltpu.SemaphoreType.DMA,
            pltpu.SemaphoreType.DMA,
        ],
    )
    def agg_kernel(y_hbm, src_hbm, dst_hbm, out_hbm, acc_sh, sidx0, sidx1,
                   didx0, didx1, rows_v, sem0, sem1):
        c = lax.axis_index("core")
        s = lax.axis_index("subcore")

        @pl.loop(0, _ZCH)
        def _zrow(i):
            @pl.loop(0, _H // 16)
            def _zcol(j):
                rows_v[i, pl.ds(j * 16, 16)] = jnp.zeros((16,), _f32)

        @pl.loop(0, _NZ // _ZCH)
        def _zacc(i):
            pltpu.sync_copy(
                rows_v.at[pl.ds(0, _ZCH)],
                acc_sh.at[pl.ds(s * _NZ + i * _ZCH, _ZCH)])

        plsc.subcore_barrier()
        e0 = (c * _NS + s) * _EPT2
        sub = ((sidx0, didx0, 0, _WA), (sidx1, didx1, _WA, _WB))

        @pl.loop(0, _EPT2 // _W)
        def _win(k):
            handles = []
            for b, (sidx, didx, roff, w) in enumerate(sub):
                off = e0 + k * _W + roff
                rows_b = rows_v.at[pl.ds(roff, w)]
                pltpu.sync_copy(src_hbm.at[pl.ds(off, w)], sidx.at[0])
                pltpu.sync_copy(dst_hbm.at[pl.ds(off, w)], didx.at[0])
                pltpu.sync_copy(y_hbm.at[sidx.at[0]], rows_b)
                handles.append(pltpu.async_copy(
                    rows_b, acc_sh.at[didx.at[0]], (sem0, sem1)[b],
                    add=True))
            for h in handles:
                h.wait()

        plsc.subcore_barrier()
        pltpu.sync_copy(acc_sh.at[pl.ds(s * _NZ, _NZ)],
                        out_hbm.at[c, pl.ds(s * _NZ, _NZ)])

    return agg_kernel(y, src, dst)


def kernel(node_features, edge_index, text_embeddings, W_proj, b_proj,
           Wg, W1, b1, W2, b2):
    src = edge_index[0]
    dst = edge_index[1]
    deg_o, deg_i = _deg_call(src, dst)
    dego = deg_o[:_N].reshape(_N, 1)
    degi = deg_i[:_N].reshape(_N, 1)

    pooled = _pool_call(text_embeddings)
    W_all = _hyper_call(pooled, Wg)
    W_all = W_all.reshape(_NL, _H, _H)

    y = _layer0_call(node_features, W_proj, b_proj.reshape(1, _H),
                     dego, W_all[0])
    for l in range(1, _NL):
        agg = _agg_call(y, src, dst)
        y = _layermid_call(agg[:, :_N], dego, degi, W_all[l])
    agg = _agg_call(y, src, dst)

    return _head_call(agg[:, :_N], degi, W1, b1.reshape(1, _HH),
                      W2, b2.reshape(1, 1))


# speedup vs baseline: 1.6412x; 1.6412x over previous
import jax
import jax.numpy as jnp
from jax import lax
from jax.experimental import pallas as pl
from jax.experimental.pallas import tpu as pltpu
from jax.experimental.pallas import tpu_sc as plsc

_N = 10000
_E = 320000
_H = 128
_HH = 64
_TD = 384
_NL = 3
_NS = 16
_NPAD = 10240
_NZ = _NPAD // _NS
_EPT = _E // _NS
_DW = 800
_W = 128
_NWIN = _E // _W
_NWT = _NWIN // 32
_NTAIL = _NWIN - 32 * _NWT
_ZCH = 128
_RB = 1000

_f32 = jnp.float32


def _pool_body(te_ref, o_ref):
    o_ref[...] = jnp.mean(te_ref[...], axis=0, keepdims=True)


def _hyper_body(p_ref, wg_ref, o_ref):
    o_ref[0] = jnp.dot(p_ref[...], wg_ref[0],
                       preferred_element_type=_f32)


def _layer0_body(nf_ref, wp_ref, bp_ref, dego_ref, w_ref, o_ref):
    x = jnp.dot(nf_ref[...], wp_ref[...], preferred_element_type=_f32)
    x = x + bp_ref[...]
    s = lax.rsqrt(jnp.maximum(dego_ref[...], 1.0))
    o_ref[...] = jnp.dot(x * s, w_ref[...], preferred_element_type=_f32)


def _layermid_body(agg_ref, dego_ref, degi_ref, w_ref, o_ref):
    x = agg_ref[0] + agg_ref[1]
    t = lax.rsqrt(jnp.maximum(degi_ref[...], 1.0))
    s = lax.rsqrt(jnp.maximum(dego_ref[...], 1.0))
    x = jnp.maximum(x * t, 0.0) * s
    o_ref[...] = jnp.dot(x, w_ref[...], preferred_element_type=_f32)


def _head_body(agg_ref, degi_ref, w1_ref, b1_ref, w2_ref, b2_ref, o_ref):
    x = agg_ref[0] + agg_ref[1]
    t = lax.rsqrt(jnp.maximum(degi_ref[...], 1.0))
    x = x * t
    h = jnp.dot(x, w1_ref[...], preferred_element_type=_f32) + b1_ref[...]
    h = jnp.maximum(h, 0.0)
    o_ref[...] = jnp.dot(h, w2_ref[...], preferred_element_type=_f32) + b2_ref[...]


def _pool_call(te):
    return pl.pallas_call(
        _pool_body,
        out_shape=jax.ShapeDtypeStruct((1, _TD), _f32),
    )(te)


def _hyper_call(pooled, Wg):
    return pl.pallas_call(
        _hyper_body,
        grid=(_NL, 8),
        in_specs=[
            pl.BlockSpec((1, _TD), lambda l, j: (0, 0)),
            pl.BlockSpec((1, _TD, 2048), lambda l, j: (l, 0, j)),
        ],
        out_specs=pl.BlockSpec((1, 1, 2048), lambda l, j: (l, 0, j)),
        out_shape=jax.ShapeDtypeStruct((_NL, 1, _H * _H), _f32),
    )(pooled, Wg)


def _layer0_call(nf, Wp, bp, dego, W0):
    return pl.pallas_call(
        _layer0_body,
        grid=(_N // _RB,),
        in_specs=[
            pl.BlockSpec((_RB, _H), lambda i: (i, 0)),
            pl.BlockSpec((_H, _H), lambda i: (0, 0)),
            pl.BlockSpec((1, _H), lambda i: (0, 0)),
            pl.BlockSpec((_RB, 1), lambda i: (i, 0)),
            pl.BlockSpec((_H, _H), lambda i: (0, 0)),
        ],
        out_specs=pl.BlockSpec((_RB, _H), lambda i: (i, 0)),
        out_shape=jax.ShapeDtypeStruct((_N, _H), _f32),
    )(nf, Wp, bp, dego, W0)


def _layermid_call(agg, dego, degi, Wl):
    return pl.pallas_call(
        _layermid_body,
        grid=(_N // _RB,),
        in_specs=[
            pl.BlockSpec((2, _RB, _H), lambda i: (0, i, 0)),
            pl.BlockSpec((_RB, 1), lambda i: (i, 0)),
            pl.BlockSpec((_RB, 1), lambda i: (i, 0)),
            pl.BlockSpec((_H, _H), lambda i: (0, 0)),
        ],
        out_specs=pl.BlockSpec((_RB, _H), lambda i: (i, 0)),
        out_shape=jax.ShapeDtypeStruct((_N, _H), _f32),
    )(agg, dego, degi, Wl)


def _head_call(agg, degi, W1, b1, W2, b2):
    return pl.pallas_call(
        _head_body,
        grid=(_N // _RB,),
        in_specs=[
            pl.BlockSpec((2, _RB, _H), lambda i: (0, i, 0)),
            pl.BlockSpec((_RB, 1), lambda i: (i, 0)),
            pl.BlockSpec((_H, _HH), lambda i: (0, 0)),
            pl.BlockSpec((1, _HH), lambda i: (0, 0)),
            pl.BlockSpec((_HH, 1), lambda i: (0, 0)),
            pl.BlockSpec((1, 1), lambda i: (0, 0)),
        ],
        out_specs=pl.BlockSpec((_RB, 1), lambda i: (i, 0)),
        out_shape=jax.ShapeDtypeStruct((_N, 1), _f32),
    )(agg, degi, W1, b1, W2, b2)


def _vector_mesh():
    return plsc.VectorSubcoreMesh(
        core_axis_name="core", subcore_axis_name="subcore")


def _deg_call(src, dst):
    @pl.kernel(
        out_type=[jax.ShapeDtypeStruct((_NPAD,), _f32),
                  jax.ShapeDtypeStruct((_NPAD,), _f32)],
        mesh=_vector_mesh(),
        scratch_types=[
            pltpu.VMEM_SHARED((_NPAD,), _f32),
            pltpu.VMEM((1, _DW), jnp.int32),
            pltpu.VMEM((1, _DW), _f32),
        ],
    )
    def deg_kernel(src_hbm, dst_hbm, out0_hbm, out1_hbm, acc_sh, idx_v, val_v):
        c = lax.axis_index("core")
        s = lax.axis_index("subcore")

        @pl.loop(0, _DW // 16)
        def _zero(i):
            val_v[0, pl.ds(i * 16, 16)] = jnp.zeros((16,), _f32)

        pltpu.sync_copy(val_v.at[0, pl.ds(0, _NZ)],
                        acc_sh.at[pl.ds(s * _NZ, _NZ)])
        plsc.subcore_barrier()

        @pl.loop(0, _DW // 16)
        def _ones(i):
            val_v[0, pl.ds(i * 16, 16)] = jnp.ones((16,), _f32)

        @pl.loop(0, _EPT // _DW)
        def _win(k):
            off = s * _EPT + k * _DW

            @pl.when(c == 0)
            def _():
                pltpu.sync_copy(src_hbm.at[pl.ds(off, _DW)], idx_v.at[0])

            @pl.when(c == 1)
            def _():
                pltpu.sync_copy(dst_hbm.at[pl.ds(off, _DW)], idx_v.at[0])

            pltpu.sync_copy(val_v.at[0], acc_sh.at[idx_v.at[0]], add=True)

        plsc.subcore_barrier()

        @pl.when(c == 0)
        def _():
            pltpu.sync_copy(acc_sh.at[pl.ds(s * _NZ, _NZ)],
                            out0_hbm.at[pl.ds(s * _NZ, _NZ)])

        @pl.when(c == 1)
        def _():
            pltpu.sync_copy(acc_sh.at[pl.ds(s * _NZ, _NZ)],
                            out1_hbm.at[pl.ds(s * _NZ, _NZ)])

    return deg_kernel(src, dst)


def _agg_call(y, idx2):
    @pl.kernel(
        out_type=jax.ShapeDtypeStruct((2, _NPAD, _H), _f32),
        mesh=_vector_mesh(),
        scratch_types=[
            pltpu.VMEM_SHARED((_NPAD, _H), _f32),
            pltpu.VMEM((2, _W), jnp.int32),
            pltpu.VMEM((2, _W), jnp.int32),
            pltpu.VMEM((2, _W), jnp.int32),
            pltpu.VMEM((_W, _H), _f32),
            pltpu.VMEM((_W, _H), _f32),
            pltpu.SemaphoreType.DMA,
            pltpu.SemaphoreType.DMA,
            pltpu.SemaphoreType.DMA,
            pltpu.SemaphoreType.DMA,
            pltpu.SemaphoreType.DMA,
        ],
    )
    def agg_kernel(y_hbm, idx2_hbm, out_hbm, acc_sh, ib0, ib1, ib2,
                   rows0, rows1, si0, si1, si2, ss0, ss1):
        c = lax.axis_index("core")
        s = lax.axis_index("subcore")
        tid = c * _NS + s
        w0 = tid * _NWT
        ibuf = (ib0, ib1, ib2)
        semi = (si0, si1, si2)
        rows = (rows0, rows1)
        sems = (ss0, ss1)

        pltpu.async_copy(idx2_hbm.at[w0], ib0, si0)
        pltpu.async_copy(idx2_hbm.at[w0 + 1], ib1, si1)

        @pl.loop(0, _ZCH)
        def _zrow(i):
            @pl.loop(0, _H // 16)
            def _zcol(j):
                rows0[i, pl.ds(j * 16, 16)] = jnp.zeros((16,), _f32)

        @pl.loop(0, _NZ // _ZCH)
        def _zacc(i):
            pltpu.sync_copy(
                rows0.at[pl.ds(0, _ZCH)],
                acc_sh.at[pl.ds(s * _NZ + i * _ZCH, _ZCH)])

        plsc.subcore_barrier()

        @pl.loop(0, _NWT, step=6)
        def _win(k0):
            for j in range(6):
                k = k0 + j
                bi = j % 3
                br = j % 2
                pi = (j + 2) % 3
                pr = 1 - br
                pltpu.make_async_copy(
                    idx2_hbm.at[w0 + k], ibuf[bi], semi[bi]).wait()
                pltpu.sync_copy(y_hbm.at[ibuf[bi].at[0]], rows[br])

                @pl.when(k >= 1)
                def _():
                    pltpu.make_async_copy(
                        rows[pr], acc_sh.at[ibuf[pi].at[1]],
                        sems[pr]).wait()

                @pl.when(k + 2 < _NWT)
                def _():
                    pltpu.async_copy(
                        idx2_hbm.at[w0 + k + 2], ibuf[pi], semi[pi])

                pltpu.async_copy(
                    rows[br], acc_sh.at[ibuf[bi].at[1]], sems[br], add=True)

        pltpu.make_async_copy(
            rows[(_NWT - 1) % 2], acc_sh.at[ibuf[(_NWT - 1) % 3].at[1]],
            sems[(_NWT - 1) % 2]).wait()

        @pl.when(tid < _NTAIL)
        def _tail():
            wt = 32 * _NWT + tid
            pltpu.sync_copy(idx2_hbm.at[wt], ibuf[0])
            pltpu.sync_copy(y_hbm.at[ibuf[0].at[0]], rows[0])
            pltpu.sync_copy(rows[0], acc_sh.at[ibuf[0].at[1]], add=True)

        plsc.subcore_barrier()
        pltpu.sync_copy(acc_sh.at[pl.ds(s * _NZ, _NZ)],
                        out_hbm.at[c, pl.ds(s * _NZ, _NZ)])

    return agg_kernel(y, idx2)


def kernel(node_features, edge_index, text_embeddings, W_proj, b_proj,
           Wg, W1, b1, W2, b2):
    src = edge_index[0]
    dst = edge_index[1]
    idx2 = jnp.stack([src.reshape(_NWIN, _W), dst.reshape(_NWIN, _W)],
                     axis=1)
    deg_o, deg_i = _deg_call(src, dst)
    dego = deg_o[:_N].reshape(_N, 1)
    degi = deg_i[:_N].reshape(_N, 1)

    pooled = _pool_call(text_embeddings)
    W_all = _hyper_call(pooled, Wg)
    W_all = W_all.reshape(_NL, _H, _H)

    y = _layer0_call(node_features, W_proj, b_proj.reshape(1, _H),
                     dego, W_all[0])
    for l in range(1, _NL):
        agg = _agg_call(y, idx2)
        y = _layermid_call(agg[:, :_N], dego, degi, W_all[l])
    agg = _agg_call(y, idx2)

    return _head_call(agg[:, :_N], degi, W1, b1.reshape(1, _HH),
                      W2, b2.reshape(1, 1))


# speedup vs baseline: 1.6450x; 1.0023x over previous
import jax
import jax.numpy as jnp
from jax import lax
from jax.experimental import pallas as pl
from jax.experimental.pallas import tpu as pltpu
from jax.experimental.pallas import tpu_sc as plsc

_N = 10000
_E = 320000
_H = 128
_HH = 64
_TD = 384
_NL = 3
_NS = 16
_NPAD = 10240
_NZ = _NPAD // _NS
_EPT = _E // _NS
_DW = 800
_W = 128
_NWIN = _E // _W
_NWT = _NWIN // 32
_NTAIL = _NWIN - 32 * _NWT
_ZCH = 128
_RB = 1000

_f32 = jnp.float32


def _pool_body(te_ref, o_ref):
    o_ref[...] = jnp.mean(te_ref[...], axis=0, keepdims=True)


def _hyper_body(p_ref, wg_ref, o_ref):
    o_ref[0] = jnp.dot(p_ref[...], wg_ref[0],
                       preferred_element_type=_f32)


def _layer0_body(nf_ref, wp_ref, bp_ref, dego_ref, w_ref, o_ref):
    x = jnp.dot(nf_ref[...], wp_ref[...], preferred_element_type=_f32)
    x = x + bp_ref[...]
    s = lax.rsqrt(jnp.maximum(dego_ref[...], 1.0))
    o_ref[...] = jnp.dot(x * s, w_ref[...], preferred_element_type=_f32)


def _layermid_body(agg_ref, dego_ref, degi_ref, w_ref, o_ref):
    x = agg_ref[0] + agg_ref[1]
    t = lax.rsqrt(jnp.maximum(degi_ref[...], 1.0))
    s = lax.rsqrt(jnp.maximum(dego_ref[...], 1.0))
    x = jnp.maximum(x * t, 0.0) * s
    o_ref[...] = jnp.dot(x, w_ref[...], preferred_element_type=_f32)


def _head_body(agg_ref, degi_ref, w1_ref, b1_ref, w2_ref, b2_ref, o_ref):
    x = agg_ref[0] + agg_ref[1]
    t = lax.rsqrt(jnp.maximum(degi_ref[...], 1.0))
    x = x * t
    h = jnp.dot(x, w1_ref[...], preferred_element_type=_f32) + b1_ref[...]
    h = jnp.maximum(h, 0.0)
    o_ref[...] = jnp.dot(h, w2_ref[...], preferred_element_type=_f32) + b2_ref[...]


def _pool_call(te):
    return pl.pallas_call(
        _pool_body,
        out_shape=jax.ShapeDtypeStruct((1, _TD), _f32),
    )(te)


def _hyper_call(pooled, Wg):
    return pl.pallas_call(
        _hyper_body,
        grid=(_NL, 8),
        in_specs=[
            pl.BlockSpec((1, _TD), lambda l, j: (0, 0)),
            pl.BlockSpec((1, _TD, 2048), lambda l, j: (l, 0, j)),
        ],
        out_specs=pl.BlockSpec((1, 1, 2048), lambda l, j: (l, 0, j)),
        out_shape=jax.ShapeDtypeStruct((_NL, 1, _H * _H), _f32),
    )(pooled, Wg)


def _layer0_call(nf, Wp, bp, dego, W0):
    return pl.pallas_call(
        _layer0_body,
        grid=(_N // _RB,),
        in_specs=[
            pl.BlockSpec((_RB, _H), lambda i: (i, 0)),
            pl.BlockSpec((_H, _H), lambda i: (0, 0)),
            pl.BlockSpec((1, _H), lambda i: (0, 0)),
            pl.BlockSpec((_RB, 1), lambda i: (i, 0)),
            pl.BlockSpec((_H, _H), lambda i: (0, 0)),
        ],
        out_specs=pl.BlockSpec((_RB, _H), lambda i: (i, 0)),
        out_shape=jax.ShapeDtypeStruct((_N, _H), _f32),
    )(nf, Wp, bp, dego, W0)


def _layermid_call(agg, dego, degi, Wl):
    return pl.pallas_call(
        _layermid_body,
        grid=(_N // _RB,),
        in_specs=[
            pl.BlockSpec((2, _RB, _H), lambda i: (0, i, 0)),
            pl.BlockSpec((_RB, 1), lambda i: (i, 0)),
            pl.BlockSpec((_RB, 1), lambda i: (i, 0)),
            pl.BlockSpec((_H, _H), lambda i: (0, 0)),
        ],
        out_specs=pl.BlockSpec((_RB, _H), lambda i: (i, 0)),
        out_shape=jax.ShapeDtypeStruct((_N, _H), _f32),
    )(agg, dego, degi, Wl)


def _head_call(agg, degi, W1, b1, W2, b2):
    return pl.pallas_call(
        _head_body,
        grid=(_N // _RB,),
        in_specs=[
            pl.BlockSpec((2, _RB, _H), lambda i: (0, i, 0)),
            pl.BlockSpec((_RB, 1), lambda i: (i, 0)),
            pl.BlockSpec((_H, _HH), lambda i: (0, 0)),
            pl.BlockSpec((1, _HH), lambda i: (0, 0)),
            pl.BlockSpec((_HH, 1), lambda i: (0, 0)),
            pl.BlockSpec((1, 1), lambda i: (0, 0)),
        ],
        out_specs=pl.BlockSpec((_RB, 1), lambda i: (i, 0)),
        out_shape=jax.ShapeDtypeStruct((_N, 1), _f32),
    )(agg, degi, W1, b1, W2, b2)


def _vector_mesh():
    return plsc.VectorSubcoreMesh(
        core_axis_name="core", subcore_axis_name="subcore")


def _deg_call(src, dst):
    @pl.kernel(
        out_type=[jax.ShapeDtypeStruct((_NPAD,), _f32),
                  jax.ShapeDtypeStruct((_NPAD,), _f32)],
        mesh=_vector_mesh(),
        scratch_types=[
            pltpu.VMEM_SHARED((_NPAD,), _f32),
            pltpu.VMEM((1, _DW), jnp.int32),
            pltpu.VMEM((1, _DW), _f32),
        ],
    )
    def deg_kernel(src_hbm, dst_hbm, out0_hbm, out1_hbm, acc_sh, idx_v, val_v):
        c = lax.axis_index("core")
        s = lax.axis_index("subcore")

        @pl.loop(0, _DW // 16)
        def _zero(i):
            val_v[0, pl.ds(i * 16, 16)] = jnp.zeros((16,), _f32)

        pltpu.sync_copy(val_v.at[0, pl.ds(0, _NZ)],
                        acc_sh.at[pl.ds(s * _NZ, _NZ)])
        plsc.subcore_barrier()

        @pl.loop(0, _DW // 16)
        def _ones(i):
            val_v[0, pl.ds(i * 16, 16)] = jnp.ones((16,), _f32)

        @pl.loop(0, _EPT // _DW)
        def _win(k):
            off = s * _EPT + k * _DW

            @pl.when(c == 0)
            def _():
                pltpu.sync_copy(src_hbm.at[pl.ds(off, _DW)], idx_v.at[0])

            @pl.when(c == 1)
            def _():
                pltpu.sync_copy(dst_hbm.at[pl.ds(off, _DW)], idx_v.at[0])

            pltpu.sync_copy(val_v.at[0], acc_sh.at[idx_v.at[0]], add=True)

        plsc.subcore_barrier()

        @pl.when(c == 0)
        def _():
            pltpu.sync_copy(acc_sh.at[pl.ds(s * _NZ, _NZ)],
                            out0_hbm.at[pl.ds(s * _NZ, _NZ)])

        @pl.when(c == 1)
        def _():
            pltpu.sync_copy(acc_sh.at[pl.ds(s * _NZ, _NZ)],
                            out1_hbm.at[pl.ds(s * _NZ, _NZ)])

    return deg_kernel(src, dst)


def _agg_call(y, idx2):
    @pl.kernel(
        out_type=jax.ShapeDtypeStruct((2, _NPAD, _H), _f32),
        mesh=_vector_mesh(),
        scratch_types=[
            pltpu.VMEM_SHARED((_NPAD, _H), _f32),
            pltpu.VMEM((2, _W), jnp.int32),
            pltpu.VMEM((2, _W), jnp.int32),
            pltpu.VMEM((2, _W), jnp.int32),
            pltpu.VMEM((2, _W), jnp.int32),
            pltpu.VMEM((_W, _H), _f32),
            pltpu.VMEM((_W, _H), _f32),
            pltpu.SemaphoreType.DMA,
            pltpu.SemaphoreType.DMA,
            pltpu.SemaphoreType.DMA,
            pltpu.SemaphoreType.DMA,
            pltpu.SemaphoreType.DMA,
            pltpu.SemaphoreType.DMA,
        ],
    )
    def agg_kernel(y_hbm, idx2_hbm, out_hbm, acc_sh, ib0, ib1, ib2, ib3,
                   rows0, rows1, si0, si1, si2, si3, ss0, ss1):
        c = lax.axis_index("core")
        s = lax.axis_index("subcore")
        tid = c * _NS + s
        w0 = tid * _NWT
        ibuf = (ib0, ib1, ib2, ib3)
        semi = (si0, si1, si2, si3)
        rows = (rows0, rows1)
        sems = (ss0, ss1)

        pltpu.async_copy(idx2_hbm.at[w0], ib0, si0)
        pltpu.async_copy(idx2_hbm.at[w0 + 1], ib1, si1)

        @pl.loop(0, _ZCH)
        def _zrow(i):
            @pl.loop(0, _H // 16)
            def _zcol(j):
                rows0[i, pl.ds(j * 16, 16)] = jnp.zeros((16,), _f32)

        @pl.loop(0, _NZ // _ZCH)
        def _zacc(i):
            pltpu.sync_copy(
                rows0.at[pl.ds(0, _ZCH)],
                acc_sh.at[pl.ds(s * _NZ + i * _ZCH, _ZCH)])

        plsc.subcore_barrier()

        def win_body(k, j, in_loop):
            bi = j % 4
            br = j % 2
            fs = (j + 2) % 4
            pltpu.make_async_copy(
                idx2_hbm.at[w0 + k], ibuf[bi], semi[bi]).wait()

            def drain_km2():
                pltpu.make_async_copy(
                    rows[br], acc_sh.at[ibuf[fs].at[1]], sems[br]).wait()

            if in_loop:
                @pl.when(k >= 2)
                def _():
                    drain_km2()

                pltpu.async_copy(idx2_hbm.at[w0 + k + 2], ibuf[fs],
                                 semi[fs])
            else:
                drain_km2()
            pltpu.sync_copy(y_hbm.at[ibuf[bi].at[0]], rows[br])
            pltpu.async_copy(
                rows[br], acc_sh.at[ibuf[bi].at[1]], sems[br], add=True)

        @pl.loop(0, _NWT - 2, step=4)
        def _win(k0):
            for j in range(4):
                win_body(k0 + j, j, True)

        win_body(_NWT - 2, (_NWT - 2) % 4, False)
        win_body(_NWT - 1, (_NWT - 1) % 4, False)

        pltpu.make_async_copy(
            rows[(_NWT - 2) % 2], acc_sh.at[ibuf[(_NWT - 2) % 4].at[1]],
            sems[(_NWT - 2) % 2]).wait()
        pltpu.make_async_copy(
            rows[(_NWT - 1) % 2], acc_sh.at[ibuf[(_NWT - 1) % 4].at[1]],
            sems[(_NWT - 1) % 2]).wait()

        @pl.when(tid < _NTAIL)
        def _tail():
            wt = 32 * _NWT + tid
            pltpu.sync_copy(idx2_hbm.at[wt], ibuf[0])
            pltpu.sync_copy(y_hbm.at[ibuf[0].at[0]], rows[0])
            pltpu.sync_copy(rows[0], acc_sh.at[ibuf[0].at[1]], add=True)

        plsc.subcore_barrier()
        pltpu.sync_copy(acc_sh.at[pl.ds(s * _NZ, _NZ)],
                        out_hbm.at[c, pl.ds(s * _NZ, _NZ)])

    return agg_kernel(y, idx2)


def kernel(node_features, edge_index, text_embeddings, W_proj, b_proj,
           Wg, W1, b1, W2, b2):
    src = edge_index[0]
    dst = edge_index[1]
    idx2 = jnp.stack([src.reshape(_NWIN, _W), dst.reshape(_NWIN, _W)],
                     axis=1)
    deg_o, deg_i = _deg_call(src, dst)
    dego = deg_o[:_N].reshape(_N, 1)
    degi = deg_i[:_N].reshape(_N, 1)

    pooled = _pool_call(text_embeddings)
    W_all = _hyper_call(pooled, Wg)
    W_all = W_all.reshape(_NL, _H, _H)

    y = _layer0_call(node_features, W_proj, b_proj.reshape(1, _H),
                     dego, W_all[0])
    for l in range(1, _NL):
        agg = _agg_call(y, idx2)
        y = _layermid_call(agg[:, :_N], dego, degi, W_all[l])
    agg = _agg_call(y, idx2)

    return _head_call(agg[:, :_N], degi, W1, b1.reshape(1, _HH),
                      W2, b2.reshape(1, 1))


# speedup vs baseline: 1.6458x; 1.0005x over previous
import jax
import jax.numpy as jnp
from jax import lax
from jax.experimental import pallas as pl
from jax.experimental.pallas import tpu as pltpu
from jax.experimental.pallas import tpu_sc as plsc

_N = 10000
_E = 320000
_H = 128
_HH = 64
_TD = 384
_NL = 3
_NS = 16
_NPAD = 10240
_NZ = _NPAD // _NS
_EPT = _E // _NS
_DW = 800
_W = 128
_NWIN = _E // _W
_NWT = _NWIN // 32
_NTAIL = _NWIN - 32 * _NWT
_ZCH = 128
_RB = 1000

_f32 = jnp.float32


def _pool_body(te_ref, o_ref):
    o_ref[...] = jnp.mean(te_ref[...], axis=0, keepdims=True)


def _hyper_body(p_ref, wg_ref, o_ref):
    o_ref[0] = jnp.dot(p_ref[...], wg_ref[0],
                       preferred_element_type=_f32)


def _layer0_body(nf_ref, wp_ref, bp_ref, dego_ref, w_ref, o_ref):
    x = jnp.dot(nf_ref[...], wp_ref[...], preferred_element_type=_f32)
    x = x + bp_ref[...]
    s = lax.rsqrt(jnp.maximum(dego_ref[...], 1.0))
    o_ref[...] = jnp.dot(x * s, w_ref[...], preferred_element_type=_f32)


def _layermid_body(agg_ref, dego_ref, degi_ref, w_ref, o_ref):
    x = agg_ref[0] + agg_ref[1]
    t = lax.rsqrt(jnp.maximum(degi_ref[...], 1.0))
    s = lax.rsqrt(jnp.maximum(dego_ref[...], 1.0))
    x = jnp.maximum(x * t, 0.0) * s
    o_ref[...] = jnp.dot(x, w_ref[...], preferred_element_type=_f32)


def _head_body(agg_ref, degi_ref, w1_ref, b1_ref, w2_ref, b2_ref, o_ref):
    x = agg_ref[0] + agg_ref[1]
    t = lax.rsqrt(jnp.maximum(degi_ref[...], 1.0))
    x = x * t
    h = jnp.dot(x, w1_ref[...], preferred_element_type=_f32) + b1_ref[...]
    h = jnp.maximum(h, 0.0)
    o_ref[...] = jnp.dot(h, w2_ref[...], preferred_element_type=_f32) + b2_ref[...]


def _pool_call(te):
    return pl.pallas_call(
        _pool_body,
        out_shape=jax.ShapeDtypeStruct((1, _TD), _f32),
    )(te)


def _hyper_call(pooled, Wg):
    return pl.pallas_call(
        _hyper_body,
        grid=(_NL, 8),
        in_specs=[
            pl.BlockSpec((1, _TD), lambda l, j: (0, 0)),
            pl.BlockSpec((1, _TD, 2048), lambda l, j: (l, 0, j)),
        ],
        out_specs=pl.BlockSpec((1, 1, 2048), lambda l, j: (l, 0, j)),
        out_shape=jax.ShapeDtypeStruct((_NL, 1, _H * _H), _f32),
    )(pooled, Wg)


def _layer0_call(nf, Wp, bp, dego, W0):
    return pl.pallas_call(
        _layer0_body,
        grid=(_N // _RB,),
        in_specs=[
            pl.BlockSpec((_RB, _H), lambda i: (i, 0)),
            pl.BlockSpec((_H, _H), lambda i: (0, 0)),
            pl.BlockSpec((1, _H), lambda i: (0, 0)),
            pl.BlockSpec((_RB, 1), lambda i: (i, 0)),
            pl.BlockSpec((_H, _H), lambda i: (0, 0)),
        ],
        out_specs=pl.BlockSpec((_RB, _H), lambda i: (i, 0)),
        out_shape=jax.ShapeDtypeStruct((_N, _H), _f32),
    )(nf, Wp, bp, dego, W0)


def _layermid_call(agg, dego, degi, Wl):
    return pl.pallas_call(
        _layermid_body,
        grid=(_N // _RB,),
        in_specs=[
            pl.BlockSpec((2, _RB, _H), lambda i: (0, i, 0)),
            pl.BlockSpec((_RB, 1), lambda i: (i, 0)),
            pl.BlockSpec((_RB, 1), lambda i: (i, 0)),
            pl.BlockSpec((_H, _H), lambda i: (0, 0)),
        ],
        out_specs=pl.BlockSpec((_RB, _H), lambda i: (i, 0)),
        out_shape=jax.ShapeDtypeStruct((_N, _H), _f32),
    )(agg, dego, degi, Wl)


def _head_call(agg, degi, W1, b1, W2, b2):
    return pl.pallas_call(
        _head_body,
        grid=(_N // _RB,),
        in_specs=[
            pl.BlockSpec((2, _RB, _H), lambda i: (0, i, 0)),
            pl.BlockSpec((_RB, 1), lambda i: (i, 0)),
            pl.BlockSpec((_H, _HH), lambda i: (0, 0)),
            pl.BlockSpec((1, _HH), lambda i: (0, 0)),
            pl.BlockSpec((_HH, 1), lambda i: (0, 0)),
            pl.BlockSpec((1, 1), lambda i: (0, 0)),
        ],
        out_specs=pl.BlockSpec((_RB, 1), lambda i: (i, 0)),
        out_shape=jax.ShapeDtypeStruct((_N, 1), _f32),
    )(agg, degi, W1, b1, W2, b2)


def _vector_mesh():
    return plsc.VectorSubcoreMesh(
        core_axis_name="core", subcore_axis_name="subcore")


def _deg_call(src, dst):
    @pl.kernel(
        out_type=[jax.ShapeDtypeStruct((_NPAD,), _f32),
                  jax.ShapeDtypeStruct((_NPAD,), _f32)],
        mesh=_vector_mesh(),
        scratch_types=[
            pltpu.VMEM_SHARED((_NPAD,), _f32),
            pltpu.VMEM((1, _DW), jnp.int32),
            pltpu.VMEM((1, _DW), _f32),
        ],
    )
    def deg_kernel(src_hbm, dst_hbm, out0_hbm, out1_hbm, acc_sh, idx_v, val_v):
        c = lax.axis_index("core")
        s = lax.axis_index("subcore")

        @pl.loop(0, _DW // 16)
        def _zero(i):
            val_v[0, pl.ds(i * 16, 16)] = jnp.zeros((16,), _f32)

        pltpu.sync_copy(val_v.at[0, pl.ds(0, _NZ)],
                        acc_sh.at[pl.ds(s * _NZ, _NZ)])
        plsc.subcore_barrier()

        @pl.loop(0, _DW // 16)
        def _ones(i):
            val_v[0, pl.ds(i * 16, 16)] = jnp.ones((16,), _f32)

        @pl.loop(0, _EPT // _DW)
        def _win(k):
            off = s * _EPT + k * _DW

            @pl.when(c == 0)
            def _():
                pltpu.sync_copy(src_hbm.at[pl.ds(off, _DW)], idx_v.at[0])

            @pl.when(c == 1)
            def _():
                pltpu.sync_copy(dst_hbm.at[pl.ds(off, _DW)], idx_v.at[0])

            pltpu.sync_copy(val_v.at[0], acc_sh.at[idx_v.at[0]], add=True)

        plsc.subcore_barrier()

        @pl.when(c == 0)
        def _():
            pltpu.sync_copy(acc_sh.at[pl.ds(s * _NZ, _NZ)],
                            out0_hbm.at[pl.ds(s * _NZ, _NZ)])

        @pl.when(c == 1)
        def _():
            pltpu.sync_copy(acc_sh.at[pl.ds(s * _NZ, _NZ)],
                            out1_hbm.at[pl.ds(s * _NZ, _NZ)])

    return deg_kernel(src, dst)


def _agg_call(y, idx2):
    @pl.kernel(
        out_type=jax.ShapeDtypeStruct((2, _NPAD, _H), _f32),
        mesh=_vector_mesh(),
        scratch_types=[
            pltpu.VMEM_SHARED((_NPAD, _H), _f32),
            pltpu.VMEM((2, _W), jnp.int32),
            pltpu.VMEM((2, _W), jnp.int32),
            pltpu.VMEM((2, _W), jnp.int32),
            pltpu.VMEM((2, _W), jnp.int32),
            pltpu.VMEM((_W, _H), _f32),
            pltpu.VMEM((_W, _H), _f32),
            pltpu.SemaphoreType.DMA,
            pltpu.SemaphoreType.DMA,
            pltpu.SemaphoreType.DMA,
            pltpu.SemaphoreType.DMA,
            pltpu.SemaphoreType.DMA,
            pltpu.SemaphoreType.DMA,
            pltpu.SemaphoreType.DMA,
            pltpu.SemaphoreType.DMA,
        ],
    )
    def agg_kernel(y_hbm, idx2_hbm, out_hbm, acc_sh, ib0, ib1, ib2, ib3,
                   rows0, rows1, si0, si1, si2, si3, ss0, ss1, sg0, sg1):
        c = lax.axis_index("core")
        s = lax.axis_index("subcore")
        tid = c * _NS + s
        w0 = tid * _NWT
        ibuf = (ib0, ib1, ib2, ib3)
        semi = (si0, si1, si2, si3)
        rows = (rows0, rows1)
        sems = (ss0, ss1)
        semg = (sg0, sg1)

        pltpu.async_copy(idx2_hbm.at[w0], ib0, si0)
        pltpu.async_copy(idx2_hbm.at[w0 + 1], ib1, si1)

        @pl.loop(0, _ZCH)
        def _zrow(i):
            @pl.loop(0, _H // 16)
            def _zcol(j):
                rows0[i, pl.ds(j * 16, 16)] = jnp.zeros((16,), _f32)

        @pl.loop(0, _NZ // _ZCH)
        def _zacc(i):
            pltpu.sync_copy(
                rows0.at[pl.ds(0, _ZCH)],
                acc_sh.at[pl.ds(s * _NZ + i * _ZCH, _ZCH)])

        pltpu.make_async_copy(idx2_hbm.at[w0], ib0, si0).wait()
        pltpu.async_copy(y_hbm.at[ib0.at[0]], rows0, sg0)
        pltpu.async_copy(idx2_hbm.at[w0 + 2], ib2, si2)

        plsc.subcore_barrier()

        def win_body(k, j, guard_prev, has_next, has_pre):
            bi = j % 4
            br = j % 2
            ni = (j + 1) % 4
            fi = (j + 3) % 4
            nr = 1 - br
            pltpu.make_async_copy(
                y_hbm.at[ibuf[bi].at[0]], rows[br], semg[br]).wait()
            pltpu.async_copy(
                rows[br], acc_sh.at[ibuf[bi].at[1]], sems[br], add=True)

            def drain_prev():
                pltpu.make_async_copy(
                    rows[nr], acc_sh.at[ibuf[fi].at[1]], sems[nr]).wait()

            if guard_prev:
                @pl.when(k >= 1)
                def _():
                    drain_prev()
            else:
                drain_prev()
            if has_next:
                pltpu.make_async_copy(
                    idx2_hbm.at[w0 + k + 1], ibuf[ni], semi[ni]).wait()
                pltpu.async_copy(
                    y_hbm.at[ibuf[ni].at[0]], rows[nr], semg[nr])
            if has_pre:
                pltpu.async_copy(idx2_hbm.at[w0 + k + 3], ibuf[fi],
                                 semi[fi])

        @pl.loop(0, _NWT - 2, step=4)
        def _win(k0):
            for j in range(4):
                win_body(k0 + j, j, True, True, True)

        win_body(_NWT - 2, (_NWT - 2) % 4, False, True, False)
        win_body(_NWT - 1, (_NWT - 1) % 4, False, False, False)

        pltpu.make_async_copy(
            rows[(_NWT - 1) % 2], acc_sh.at[ibuf[(_NWT - 1) % 4].at[1]],
            sems[(_NWT - 1) % 2]).wait()

        @pl.when(tid < _NTAIL)
        def _tail():
            wt = 32 * _NWT + tid
            pltpu.sync_copy(idx2_hbm.at[wt], ibuf[0])
            pltpu.sync_copy(y_hbm.at[ibuf[0].at[0]], rows[0])
            pltpu.sync_copy(rows[0], acc_sh.at[ibuf[0].at[1]], add=True)

        plsc.subcore_barrier()
        pltpu.sync_copy(acc_sh.at[pl.ds(s * _NZ, _NZ)],
                        out_hbm.at[c, pl.ds(s * _NZ, _NZ)])

    return agg_kernel(y, idx2)


def kernel(node_features, edge_index, text_embeddings, W_proj, b_proj,
           Wg, W1, b1, W2, b2):
    src = edge_index[0]
    dst = edge_index[1]
    idx2 = jnp.stack([src.reshape(_NWIN, _W), dst.reshape(_NWIN, _W)],
                     axis=1)
    deg_o, deg_i = _deg_call(src, dst)
    dego = deg_o[:_N].reshape(_N, 1)
    degi = deg_i[:_N].reshape(_N, 1)

    pooled = _pool_call(text_embeddings)
    W_all = _hyper_call(pooled, Wg)
    W_all = W_all.reshape(_NL, _H, _H)

    y = _layer0_call(node_features, W_proj, b_proj.reshape(1, _H),
                     dego, W_all[0])
    for l in range(1, _NL):
        agg = _agg_call(y, idx2)
        y = _layermid_call(agg[:, :_N], dego, degi, W_all[l])
    agg = _agg_call(y, idx2)

    return _head_call(agg[:, :_N], degi, W1, b1.reshape(1, _HH),
                      W2, b2.reshape(1, 1))


# speedup vs baseline: 1.7188x; 1.0444x over previous
import jax
import jax.numpy as jnp
from jax import lax
from jax.experimental import pallas as pl
from jax.experimental.pallas import tpu as pltpu
from jax.experimental.pallas import tpu_sc as plsc

_N = 10000
_E = 320000
_H = 128
_HH = 64
_TD = 384
_NL = 3
_NS = 16
_NPAD = 10240
_NZ = _NPAD // _NS
_EPT = _E // _NS
_DW = 800
_W = 128
_NWIN = _E // _W
_NWT = _NWIN // 32
_NTAIL = _NWIN - 32 * _NWT
_ZCH = 128
_RB = 1000

_f32 = jnp.float32


def _pool_body(te_ref, o_ref):
    o_ref[...] = jnp.mean(te_ref[...], axis=0, keepdims=True)


def _hyper_body(p_ref, wg_ref, o_ref):
    o_ref[0] = jnp.dot(p_ref[...], wg_ref[0],
                       preferred_element_type=_f32)


def _layer0_body(nf_ref, wp_ref, bp_ref, dego_ref, w_ref, o_ref):
    x = jnp.dot(nf_ref[...], wp_ref[...], preferred_element_type=_f32)
    x = x + bp_ref[...]
    s = lax.rsqrt(jnp.maximum(dego_ref[...], 1.0))
    o_ref[...] = jnp.dot(x * s, w_ref[...], preferred_element_type=_f32)


def _layermid_body(agg_ref, dego_ref, degi_ref, w_ref, o_ref):
    x = agg_ref[0] + agg_ref[1]
    t = lax.rsqrt(jnp.maximum(degi_ref[...], 1.0))
    s = lax.rsqrt(jnp.maximum(dego_ref[...], 1.0))
    x = jnp.maximum(x * t, 0.0) * s
    o_ref[...] = jnp.dot(x, w_ref[...], preferred_element_type=_f32)


def _head_body(agg_ref, degi_ref, w1_ref, b1_ref, w2_ref, b2_ref, o_ref):
    x = agg_ref[0] + agg_ref[1]
    t = lax.rsqrt(jnp.maximum(degi_ref[...], 1.0))
    x = x * t
    h = jnp.dot(x, w1_ref[...], preferred_element_type=_f32) + b1_ref[...]
    h = jnp.maximum(h, 0.0)
    o_ref[...] = jnp.dot(h, w2_ref[...], preferred_element_type=_f32) + b2_ref[...]


def _idx2_body(s_ref, d_ref, o_ref):
    o_ref[:, 0, :] = s_ref[...]
    o_ref[:, 1, :] = d_ref[...]


def _idx2_call(src, dst):
    return pl.pallas_call(
        _idx2_body,
        out_shape=jax.ShapeDtypeStruct((_NWIN, 2, _W), jnp.int32),
    )(src.reshape(_NWIN, _W), dst.reshape(_NWIN, _W))


def _pool_call(te):
    return pl.pallas_call(
        _pool_body,
        out_shape=jax.ShapeDtypeStruct((1, _TD), _f32),
    )(te)


def _hyper_call(pooled, Wg):
    return pl.pallas_call(
        _hyper_body,
        grid=(_NL, 8),
        in_specs=[
            pl.BlockSpec((1, _TD), lambda l, j: (0, 0)),
            pl.BlockSpec((1, _TD, 2048), lambda l, j: (l, 0, j)),
        ],
        out_specs=pl.BlockSpec((1, 1, 2048), lambda l, j: (l, 0, j)),
        out_shape=jax.ShapeDtypeStruct((_NL, 1, _H * _H), _f32),
    )(pooled, Wg)


def _layer0_call(nf, Wp, bp, dego, W0):
    return pl.pallas_call(
        _layer0_body,
        grid=(_N // _RB,),
        in_specs=[
            pl.BlockSpec((_RB, _H), lambda i: (i, 0)),
            pl.BlockSpec((_H, _H), lambda i: (0, 0)),
            pl.BlockSpec((1, _H), lambda i: (0, 0)),
            pl.BlockSpec((_RB, 1), lambda i: (i, 0)),
            pl.BlockSpec((_H, _H), lambda i: (0, 0)),
        ],
        out_specs=pl.BlockSpec((_RB, _H), lambda i: (i, 0)),
        out_shape=jax.ShapeDtypeStruct((_N, _H), _f32),
    )(nf, Wp, bp, dego, W0)


def _layermid_call(agg, dego, degi, Wl):
    return pl.pallas_call(
        _layermid_body,
        grid=(_N // _RB,),
        in_specs=[
            pl.BlockSpec((2, _RB, _H), lambda i: (0, i, 0)),
            pl.BlockSpec((_RB, 1), lambda i: (i, 0)),
            pl.BlockSpec((_RB, 1), lambda i: (i, 0)),
            pl.BlockSpec((_H, _H), lambda i: (0, 0)),
        ],
        out_specs=pl.BlockSpec((_RB, _H), lambda i: (i, 0)),
        out_shape=jax.ShapeDtypeStruct((_N, _H), _f32),
    )(agg, dego, degi, Wl)


def _head_call(agg, degi, W1, b1, W2, b2):
    return pl.pallas_call(
        _head_body,
        grid=(_N // _RB,),
        in_specs=[
            pl.BlockSpec((2, _RB, _H), lambda i: (0, i, 0)),
            pl.BlockSpec((_RB, 1), lambda i: (i, 0)),
            pl.BlockSpec((_H, _HH), lambda i: (0, 0)),
            pl.BlockSpec((1, _HH), lambda i: (0, 0)),
            pl.BlockSpec((_HH, 1), lambda i: (0, 0)),
            pl.BlockSpec((1, 1), lambda i: (0, 0)),
        ],
        out_specs=pl.BlockSpec((_RB, 1), lambda i: (i, 0)),
        out_shape=jax.ShapeDtypeStruct((_N, 1), _f32),
    )(agg, degi, W1, b1, W2, b2)


def _vector_mesh():
    return plsc.VectorSubcoreMesh(
        core_axis_name="core", subcore_axis_name="subcore")


def _deg_call(src, dst):
    @pl.kernel(
        out_type=[jax.ShapeDtypeStruct((_NPAD,), _f32),
                  jax.ShapeDtypeStruct((_NPAD,), _f32)],
        mesh=_vector_mesh(),
        scratch_types=[
            pltpu.VMEM_SHARED((_NPAD,), _f32),
            pltpu.VMEM((1, _DW), jnp.int32),
            pltpu.VMEM((1, _DW), _f32),
        ],
    )
    def deg_kernel(src_hbm, dst_hbm, out0_hbm, out1_hbm, acc_sh, idx_v, val_v):
        c = lax.axis_index("core")
        s = lax.axis_index("subcore")

        @pl.loop(0, _DW // 16)
        def _zero(i):
            val_v[0, pl.ds(i * 16, 16)] = jnp.zeros((16,), _f32)

        pltpu.sync_copy(val_v.at[0, pl.ds(0, _NZ)],
                        acc_sh.at[pl.ds(s * _NZ, _NZ)])
        plsc.subcore_barrier()

        @pl.loop(0, _DW // 16)
        def _ones(i):
            val_v[0, pl.ds(i * 16, 16)] = jnp.ones((16,), _f32)

        @pl.loop(0, _EPT // _DW)
        def _win(k):
            off = s * _EPT + k * _DW

            @pl.when(c == 0)
            def _():
                pltpu.sync_copy(src_hbm.at[pl.ds(off, _DW)], idx_v.at[0])

            @pl.when(c == 1)
            def _():
                pltpu.sync_copy(dst_hbm.at[pl.ds(off, _DW)], idx_v.at[0])

            pltpu.sync_copy(val_v.at[0], acc_sh.at[idx_v.at[0]], add=True)

        plsc.subcore_barrier()

        @pl.when(c == 0)
        def _():
            pltpu.sync_copy(acc_sh.at[pl.ds(s * _NZ, _NZ)],
                            out0_hbm.at[pl.ds(s * _NZ, _NZ)])

        @pl.when(c == 1)
        def _():
            pltpu.sync_copy(acc_sh.at[pl.ds(s * _NZ, _NZ)],
                            out1_hbm.at[pl.ds(s * _NZ, _NZ)])

    return deg_kernel(src, dst)


def _agg_call(y, idx2):
    @pl.kernel(
        out_type=jax.ShapeDtypeStruct((2, _NPAD, _H), _f32),
        mesh=_vector_mesh(),
        scratch_types=[
            pltpu.VMEM_SHARED((_NPAD, _H), _f32),
            pltpu.VMEM((2, _W), jnp.int32),
            pltpu.VMEM((2, _W), jnp.int32),
            pltpu.VMEM((2, _W), jnp.int32),
            pltpu.VMEM((2, _W), jnp.int32),
            pltpu.VMEM((_W, _H), _f32),
            pltpu.VMEM((_W, _H), _f32),
            pltpu.SemaphoreType.DMA,
            pltpu.SemaphoreType.DMA,
            pltpu.SemaphoreType.DMA,
            pltpu.SemaphoreType.DMA,
            pltpu.SemaphoreType.DMA,
            pltpu.SemaphoreType.DMA,
            pltpu.SemaphoreType.DMA,
            pltpu.SemaphoreType.DMA,
        ],
    )
    def agg_kernel(y_hbm, idx2_hbm, out_hbm, acc_sh, ib0, ib1, ib2, ib3,
                   rows0, rows1, si0, si1, si2, si3, ss0, ss1, sg0, sg1):
        c = lax.axis_index("core")
        s = lax.axis_index("subcore")
        tid = c * _NS + s
        w0 = tid * _NWT
        ibuf = (ib0, ib1, ib2, ib3)
        semi = (si0, si1, si2, si3)
        rows = (rows0, rows1)
        sems = (ss0, ss1)
        semg = (sg0, sg1)

        pltpu.async_copy(idx2_hbm.at[w0], ib0, si0)
        pltpu.async_copy(idx2_hbm.at[w0 + 1], ib1, si1)

        @pl.loop(0, _ZCH)
        def _zrow(i):
            @pl.loop(0, _H // 16)
            def _zcol(j):
                rows0[i, pl.ds(j * 16, 16)] = jnp.zeros((16,), _f32)

        @pl.loop(0, _NZ // _ZCH)
        def _zacc(i):
            pltpu.sync_copy(
                rows0.at[pl.ds(0, _ZCH)],
                acc_sh.at[pl.ds(s * _NZ + i * _ZCH, _ZCH)])

        pltpu.make_async_copy(idx2_hbm.at[w0], ib0, si0).wait()
        pltpu.async_copy(y_hbm.at[ib0.at[0]], rows0, sg0)
        pltpu.async_copy(idx2_hbm.at[w0 + 2], ib2, si2)

        plsc.subcore_barrier()

        def win_body(k, j, guard_prev, has_next, has_pre):
            bi = j % 4
            br = j % 2
            ni = (j + 1) % 4
            fi = (j + 3) % 4
            nr = 1 - br
            pltpu.make_async_copy(
                y_hbm.at[ibuf[bi].at[0]], rows[br], semg[br]).wait()
            pltpu.async_copy(
                rows[br], acc_sh.at[ibuf[bi].at[1]], sems[br], add=True)

            def drain_prev():
                pltpu.make_async_copy(
                    rows[nr], acc_sh.at[ibuf[fi].at[1]], sems[nr]).wait()

            if guard_prev:
                @pl.when(k >= 1)
                def _():
                    drain_prev()
            else:
                drain_prev()
            if has_next:
                pltpu.make_async_copy(
                    idx2_hbm.at[w0 + k + 1], ibuf[ni], semi[ni]).wait()
                pltpu.async_copy(
                    y_hbm.at[ibuf[ni].at[0]], rows[nr], semg[nr])
            if has_pre:
                pltpu.async_copy(idx2_hbm.at[w0 + k + 3], ibuf[fi],
                                 semi[fi])

        @pl.loop(0, _NWT - 2, step=4)
        def _win(k0):
            for j in range(4):
                win_body(k0 + j, j, True, True, True)

        win_body(_NWT - 2, (_NWT - 2) % 4, False, True, False)
        win_body(_NWT - 1, (_NWT - 1) % 4, False, False, False)

        pltpu.make_async_copy(
            rows[(_NWT - 1) % 2], acc_sh.at[ibuf[(_NWT - 1) % 4].at[1]],
            sems[(_NWT - 1) % 2]).wait()

        @pl.when(tid < _NTAIL)
        def _tail():
            wt = 32 * _NWT + tid
            pltpu.sync_copy(idx2_hbm.at[wt], ibuf[0])
            pltpu.sync_copy(y_hbm.at[ibuf[0].at[0]], rows[0])
            pltpu.sync_copy(rows[0], acc_sh.at[ibuf[0].at[1]], add=True)

        plsc.subcore_barrier()
        pltpu.sync_copy(acc_sh.at[pl.ds(s * _NZ, _NZ)],
                        out_hbm.at[c, pl.ds(s * _NZ, _NZ)])

    return agg_kernel(y, idx2)


def kernel(node_features, edge_index, text_embeddings, W_proj, b_proj,
           Wg, W1, b1, W2, b2):
    src = edge_index[0]
    dst = edge_index[1]
    idx2 = _idx2_call(src, dst)
    deg_o, deg_i = _deg_call(src, dst)
    dego = deg_o[:_N].reshape(_N, 1)
    degi = deg_i[:_N].reshape(_N, 1)

    pooled = _pool_call(text_embeddings)
    W_all = _hyper_call(pooled, Wg)
    W_all = W_all.reshape(_NL, _H, _H)

    y = _layer0_call(node_features, W_proj, b_proj.reshape(1, _H),
                     dego, W_all[0])
    for l in range(1, _NL):
        agg = _agg_call(y, idx2)
        y = _layermid_call(agg, dego, degi, W_all[l])
    agg = _agg_call(y, idx2)

    return _head_call(agg, degi, W1, b1.reshape(1, _HH),
                      W2, b2.reshape(1, 1))


# speedup vs baseline: 1.7322x; 1.0078x over previous
import jax
import jax.numpy as jnp
from jax import lax
from jax.experimental import pallas as pl
from jax.experimental.pallas import tpu as pltpu
from jax.experimental.pallas import tpu_sc as plsc

_N = 10000
_E = 320000
_H = 128
_HH = 64
_TD = 384
_NL = 3
_NS = 16
_NPAD = 10240
_NZ = _NPAD // _NS
_EPT = _E // _NS
_DW = 800
_W = 128
_NWIN = _E // _W
_NWT = _NWIN // 32
_NTAIL = _NWIN - 32 * _NWT
_ZCH = 128
_RB = 1000

_f32 = jnp.float32


def _pool_body(te_ref, o_ref):
    o_ref[...] = jnp.mean(te_ref[...], axis=0, keepdims=True)


def _hyper_body(p_ref, wg_ref, o_ref):
    o_ref[0] = jnp.dot(p_ref[...], wg_ref[0],
                       preferred_element_type=_f32)


def _layer0_body(nf_ref, wp_ref, bp_ref, dego_ref, w_ref, o_ref):
    x = jnp.dot(nf_ref[...], wp_ref[...], preferred_element_type=_f32)
    x = x + bp_ref[...]
    s = lax.rsqrt(jnp.maximum(dego_ref[...], 1.0))
    o_ref[...] = jnp.dot(x * s, w_ref[...], preferred_element_type=_f32)


def _layermid_body(agg_ref, dego_ref, degi_ref, w_ref, o_ref):
    x = agg_ref[0] + agg_ref[1]
    t = lax.rsqrt(jnp.maximum(degi_ref[...], 1.0))
    s = lax.rsqrt(jnp.maximum(dego_ref[...], 1.0))
    x = jnp.maximum(x * t, 0.0) * s
    o_ref[...] = jnp.dot(x, w_ref[...], preferred_element_type=_f32)


def _head_body(agg_ref, degi_ref, w1_ref, b1_ref, w2_ref, b2_ref, o_ref):
    x = agg_ref[0] + agg_ref[1]
    t = lax.rsqrt(jnp.maximum(degi_ref[...], 1.0))
    x = x * t
    h = jnp.dot(x, w1_ref[...], preferred_element_type=_f32) + b1_ref[...]
    h = jnp.maximum(h, 0.0)
    o_ref[...] = jnp.dot(h, w2_ref[...], preferred_element_type=_f32) + b2_ref[...]


def _idx2_body(s_ref, d_ref, o_ref):
    o_ref[:, 0, :] = s_ref[...]
    o_ref[:, 1, :] = d_ref[...]


def _idx2_call(src, dst):
    return pl.pallas_call(
        _idx2_body,
        out_shape=jax.ShapeDtypeStruct((_NWIN, 2, _W), jnp.int32),
    )(src.reshape(_NWIN, _W), dst.reshape(_NWIN, _W))


def _pool_call(te):
    return pl.pallas_call(
        _pool_body,
        out_shape=jax.ShapeDtypeStruct((1, _TD), _f32),
    )(te)


def _hyper_call(pooled, Wg, l0, nl):
    return pl.pallas_call(
        _hyper_body,
        grid=(nl, 8),
        in_specs=[
            pl.BlockSpec((1, _TD), lambda l, j: (0, 0)),
            pl.BlockSpec((1, _TD, 2048), lambda l, j: (l + l0, 0, j)),
        ],
        out_specs=pl.BlockSpec((1, 1, 2048), lambda l, j: (l, 0, j)),
        out_shape=jax.ShapeDtypeStruct((nl, 1, _H * _H), _f32),
    )(pooled, Wg)


def _layer0_call(nf, Wp, bp, dego, W0):
    return pl.pallas_call(
        _layer0_body,
        grid=(_N // _RB,),
        in_specs=[
            pl.BlockSpec((_RB, _H), lambda i: (i, 0)),
            pl.BlockSpec((_H, _H), lambda i: (0, 0)),
            pl.BlockSpec((1, _H), lambda i: (0, 0)),
            pl.BlockSpec((_RB, 1), lambda i: (i, 0)),
            pl.BlockSpec((_H, _H), lambda i: (0, 0)),
        ],
        out_specs=pl.BlockSpec((_RB, _H), lambda i: (i, 0)),
        out_shape=jax.ShapeDtypeStruct((_N, _H), _f32),
    )(nf, Wp, bp, dego, W0)


def _layermid_call(agg, dego, degi, Wl):
    return pl.pallas_call(
        _layermid_body,
        grid=(_N // _RB,),
        in_specs=[
            pl.BlockSpec((2, _RB, _H), lambda i: (0, i, 0)),
            pl.BlockSpec((_RB, 1), lambda i: (i, 0)),
            pl.BlockSpec((_RB, 1), lambda i: (i, 0)),
            pl.BlockSpec((_H, _H), lambda i: (0, 0)),
        ],
        out_specs=pl.BlockSpec((_RB, _H), lambda i: (i, 0)),
        out_shape=jax.ShapeDtypeStruct((_N, _H), _f32),
    )(agg, dego, degi, Wl)


def _head_call(agg, degi, W1, b1, W2, b2):
    return pl.pallas_call(
        _head_body,
        grid=(_N // _RB,),
        in_specs=[
            pl.BlockSpec((2, _RB, _H), lambda i: (0, i, 0)),
            pl.BlockSpec((_RB, 1), lambda i: (i, 0)),
            pl.BlockSpec((_H, _HH), lambda i: (0, 0)),
            pl.BlockSpec((1, _HH), lambda i: (0, 0)),
            pl.BlockSpec((_HH, 1), lambda i: (0, 0)),
            pl.BlockSpec((1, 1), lambda i: (0, 0)),
        ],
        out_specs=pl.BlockSpec((_RB, 1), lambda i: (i, 0)),
        out_shape=jax.ShapeDtypeStruct((_N, 1), _f32),
    )(agg, degi, W1, b1, W2, b2)


def _vector_mesh():
    return plsc.VectorSubcoreMesh(
        core_axis_name="core", subcore_axis_name="subcore")


def _deg_call(src, dst):
    @pl.kernel(
        out_type=[jax.ShapeDtypeStruct((_NPAD,), _f32),
                  jax.ShapeDtypeStruct((_NPAD,), _f32)],
        mesh=_vector_mesh(),
        scratch_types=[
            pltpu.VMEM_SHARED((_NPAD,), _f32),
            pltpu.VMEM((1, _DW), jnp.int32),
            pltpu.VMEM((1, _DW), _f32),
        ],
    )
    def deg_kernel(src_hbm, dst_hbm, out0_hbm, out1_hbm, acc_sh, idx_v,
                   val_v):
        c = lax.axis_index("core")
        s = lax.axis_index("subcore")

        @pl.loop(0, _DW // 16)
        def _zero(i):
            val_v[0, pl.ds(i * 16, 16)] = jnp.zeros((16,), _f32)

        pltpu.sync_copy(val_v.at[0, pl.ds(0, _NZ)],
                        acc_sh.at[pl.ds(s * _NZ, _NZ)])
        plsc.subcore_barrier()

        @pl.loop(0, _DW // 16)
        def _ones(i):
            val_v[0, pl.ds(i * 16, 16)] = jnp.ones((16,), _f32)

        @pl.loop(0, _EPT // _DW)
        def _win(k):
            off = s * _EPT + k * _DW

            @pl.when(c == 0)
            def _():
                pltpu.sync_copy(src_hbm.at[pl.ds(off, _DW)], idx_v.at[0])

            @pl.when(c == 1)
            def _():
                pltpu.sync_copy(dst_hbm.at[pl.ds(off, _DW)], idx_v.at[0])

            pltpu.sync_copy(val_v.at[0], acc_sh.at[idx_v.at[0]], add=True)

        plsc.subcore_barrier()

        @pl.when(c == 0)
        def _():
            pltpu.sync_copy(acc_sh.at[pl.ds(s * _NZ, _NZ)],
                            out0_hbm.at[pl.ds(s * _NZ, _NZ)])

        @pl.when(c == 1)
        def _():
            pltpu.sync_copy(acc_sh.at[pl.ds(s * _NZ, _NZ)],
                            out1_hbm.at[pl.ds(s * _NZ, _NZ)])

    return deg_kernel(src, dst)


def _agg_call(y, idx2):
    @pl.kernel(
        out_type=jax.ShapeDtypeStruct((2, _NPAD, _H), _f32),
        mesh=_vector_mesh(),
        scratch_types=[
            pltpu.VMEM_SHARED((_NPAD, _H), _f32),
            pltpu.VMEM((2, _W), jnp.int32),
            pltpu.VMEM((2, _W), jnp.int32),
            pltpu.VMEM((2, _W), jnp.int32),
            pltpu.VMEM((2, _W), jnp.int32),
            pltpu.VMEM((_W, _H), _f32),
            pltpu.VMEM((_W, _H), _f32),
            pltpu.SemaphoreType.DMA,
            pltpu.SemaphoreType.DMA,
            pltpu.SemaphoreType.DMA,
            pltpu.SemaphoreType.DMA,
            pltpu.SemaphoreType.DMA,
            pltpu.SemaphoreType.DMA,
            pltpu.SemaphoreType.DMA,
            pltpu.SemaphoreType.DMA,
        ],
    )
    def agg_kernel(y_hbm, idx2_hbm, out_hbm, acc_sh, ib0, ib1, ib2, ib3,
                   rows0, rows1, si0, si1, si2, si3, ss0, ss1, sg0, sg1):
        c = lax.axis_index("core")
        s = lax.axis_index("subcore")
        tid = c * _NS + s
        w0 = tid * _NWT
        ibuf = (ib0, ib1, ib2, ib3)
        semi = (si0, si1, si2, si3)
        rows = (rows0, rows1)
        sems = (ss0, ss1)
        semg = (sg0, sg1)

        pltpu.async_copy(idx2_hbm.at[w0], ib0, si0)
        pltpu.async_copy(idx2_hbm.at[w0 + 1], ib1, si1)

        @pl.loop(0, _ZCH)
        def _zrow(i):
            @pl.loop(0, _H // 16)
            def _zcol(j):
                rows0[i, pl.ds(j * 16, 16)] = jnp.zeros((16,), _f32)

        @pl.loop(0, _NZ // _ZCH)
        def _zacc(i):
            pltpu.sync_copy(
                rows0.at[pl.ds(0, _ZCH)],
                acc_sh.at[pl.ds(s * _NZ + i * _ZCH, _ZCH)])

        pltpu.make_async_copy(idx2_hbm.at[w0], ib0, si0).wait()
        pltpu.async_copy(y_hbm.at[ib0.at[0]], rows0, sg0)
        pltpu.async_copy(idx2_hbm.at[w0 + 2], ib2, si2)

        plsc.subcore_barrier()

        def win_body(k, j, guard_prev, has_next, has_pre):
            bi = j % 4
            br = j % 2
            ni = (j + 1) % 4
            fi = (j + 3) % 4
            nr = 1 - br
            pltpu.make_async_copy(
                y_hbm.at[ibuf[bi].at[0]], rows[br], semg[br]).wait()
            pltpu.async_copy(
                rows[br], acc_sh.at[ibuf[bi].at[1]], sems[br], add=True)

            def drain_prev():
                pltpu.make_async_copy(
                    rows[nr], acc_sh.at[ibuf[fi].at[1]], sems[nr]).wait()

            if guard_prev:
                @pl.when(k >= 1)
                def _():
                    drain_prev()
            else:
                drain_prev()
            if has_next:
                pltpu.make_async_copy(
                    idx2_hbm.at[w0 + k + 1], ibuf[ni], semi[ni]).wait()
                pltpu.async_copy(
                    y_hbm.at[ibuf[ni].at[0]], rows[nr], semg[nr])
            if has_pre:
                pltpu.async_copy(idx2_hbm.at[w0 + k + 3], ibuf[fi],
                                 semi[fi])

        @pl.loop(0, _NWT - 2, step=4)
        def _win(k0):
            for j in range(4):
                win_body(k0 + j, j, True, True, True)

        win_body(_NWT - 2, (_NWT - 2) % 4, False, True, False)
        win_body(_NWT - 1, (_NWT - 1) % 4, False, False, False)

        pltpu.make_async_copy(
            rows[(_NWT - 1) % 2], acc_sh.at[ibuf[(_NWT - 1) % 4].at[1]],
            sems[(_NWT - 1) % 2]).wait()

        @pl.when(tid < _NTAIL)
        def _tail():
            wt = 32 * _NWT + tid
            pltpu.sync_copy(idx2_hbm.at[wt], ibuf[0])
            pltpu.sync_copy(y_hbm.at[ibuf[0].at[0]], rows[0])
            pltpu.sync_copy(rows[0], acc_sh.at[ibuf[0].at[1]], add=True)

        plsc.subcore_barrier()
        pltpu.sync_copy(acc_sh.at[pl.ds(s * _NZ, _NZ)],
                        out_hbm.at[c, pl.ds(s * _NZ, _NZ)])

    return agg_kernel(y, idx2)


def kernel(node_features, edge_index, text_embeddings, W_proj, b_proj,
           Wg, W1, b1, W2, b2):
    src = edge_index[0]
    dst = edge_index[1]
    idx2 = _idx2_call(src, dst)
    deg_o, deg_i = _deg_call(src, dst)
    dego = deg_o[:_N].reshape(_N, 1)
    degi = deg_i[:_N].reshape(_N, 1)

    pooled = _pool_call(text_embeddings)
    W0 = _hyper_call(pooled, Wg, 0, 1).reshape(_H, _H)
    W12 = _hyper_call(pooled, Wg, 1, 2).reshape(2, _H, _H)
    W_all = [W0, W12[0], W12[1]]

    y = _layer0_call(node_features, W_proj, b_proj.reshape(1, _H),
                     dego, W_all[0])
    for l in range(1, _NL):
        agg = _agg_call(y, idx2)
        y = _layermid_call(agg, dego, degi, W_all[l])
    agg = _agg_call(y, idx2)

    return _head_call(agg, degi, W1, b1.reshape(1, _HH),
                      W2, b2.reshape(1, 1))
